# baseline (speedup 1.0000x reference)
import dataclasses
import functools

import jax
import jax.numpy as jnp
from jax import lax
from jax.experimental import pallas as pl
from jax.experimental.pallas import tpu as pltpu
from jax.experimental.pallas import tpu_sc as plsc

BATCH = 16384
EMBED_DIM = 128
NUM_GENRES = 32
HIDDEN_DIM = 1024

NC = 2
NS = 16
NW = NC * NS
BPW = BATCH // NW
CHUNK = 128
NCHUNK = BPW // CHUNK
NLANE = 16


def _sc_gather(user_emb, movie_emb, ubt2, mbt2, uidx2, midx2, nbatch):
    bpw = nbatch // NW
    nchunk = bpw // CHUNK
    mesh = plsc.VectorSubcoreMesh(core_axis_name="c", subcore_axis_name="s")
    out_types = (
        jax.ShapeDtypeStruct((nbatch, 2 * EMBED_DIM), jnp.float32),
        jax.ShapeDtypeStruct((nbatch,), jnp.float32),
    )

    cp = pltpu.CompilerParams()
    if "needs_layout_passes" in pltpu.CompilerParams.__dataclass_fields__:
        cp = dataclasses.replace(cp, needs_layout_passes=False)

    @functools.partial(
        pl.kernel,
        mesh=mesh,
        out_type=out_types,
        compiler_params=cp,
        scratch_types=[
            pltpu.VMEM((nchunk, CHUNK), jnp.int32),
            pltpu.VMEM((nchunk, CHUNK), jnp.int32),
            pltpu.VMEM((bpw, EMBED_DIM), jnp.float32),
            pltpu.VMEM((2, CHUNK, EMBED_DIM), jnp.float32),
            pltpu.VMEM((bpw,), jnp.float32),
            pltpu.SemaphoreType.DMA,
            pltpu.SemaphoreType.DMA,
        ],
    )
    def k(uemb, memb, ubt, mbt, uidx, midx, out_um, out_bias,
          idx_v, ridx_v, rows_v, bbuf, bias_v, sem, bsem):
        wid = lax.axis_index("s") * NC + lax.axis_index("c")
        base = wid * bpw
        row0 = wid * nchunk
        lanes = lax.iota(jnp.int32, NLANE)

        def gather_one(idx_hbm, table, bias_tab, col0, accumulate):
            pltpu.sync_copy(idx_hbm.at[pl.ds(row0, nchunk)], idx_v)
            row_cps = [
                pltpu.async_copy(
                    table.at[idx_v.at[j]],
                    rows_v.at[pl.ds(j * CHUNK, CHUNK)], sem)
                for j in range(nchunk)
            ]
            for j in range(nchunk):
                for b in range(CHUNK // NLANE):
                    iv = idx_v[j, pl.ds(b * NLANE, NLANE)]
                    ridx_v[j, pl.ds(b * NLANE, NLANE)] = iv >> 7
            nbuf = min(2, nchunk)
            bias_cps = [None] * nbuf
            for j in range(nbuf):
                bias_cps[j] = pltpu.async_copy(
                    bias_tab.at[ridx_v.at[j]], bbuf.at[j], bsem)
            for j in range(nchunk):
                bias_cps[j % nbuf].wait()
                for b in range(CHUNK // NLANE):
                    iv = idx_v[j, pl.ds(b * NLANE, NLANE)]
                    k_vec = lanes + (b * NLANE)
                    lane_vec = iv & 127
                    vals = plsc.load_gather(bbuf.at[j % nbuf], [k_vec, lane_vec])
                    dst = pl.ds(j * CHUNK + b * NLANE, NLANE)
                    if accumulate:
                        bias_v[dst] = bias_v[dst] + vals
                    else:
                        bias_v[dst] = vals
                if j + nbuf < nchunk:
                    bias_cps[j % nbuf] = pltpu.async_copy(
                        bias_tab.at[ridx_v.at[j + nbuf]], bbuf.at[j % nbuf],
                        bsem)
            for c in row_cps:
                c.wait()
            pltpu.sync_copy(
                rows_v,
                out_um.at[pl.ds(base, bpw), pl.ds(col0, EMBED_DIM)])

        gather_one(uidx, uemb, ubt, 0, False)
        gather_one(midx, memb, mbt, EMBED_DIM, True)
        pltpu.sync_copy(bias_v, out_bias.at[pl.ds(base, bpw)])

    return k(user_emb, movie_emb, ubt2, mbt2, uidx2, midx2)


B_BLK = 2048


def _mlp_body(um_ref, g_ref, w1_ref, b1_ref, w2_ref, bias_ref, c_ref, o_ref):
    um = um_ref[...].astype(jnp.bfloat16)
    f = jnp.concatenate([um, g_ref[...]], axis=1)
    h = jnp.dot(f, w1_ref[...], preferred_element_type=jnp.float32)
    h = jnp.maximum(h + b1_ref[...], 0.0).astype(jnp.bfloat16)
    s = jnp.dot(h, w2_ref[...], preferred_element_type=jnp.float32)
    o_ref[...] = s.reshape(B_BLK) + bias_ref[...] + c_ref[0]


def _mlp(um_rows, genre, w1, b1r, w2, bias_sum, cvec, nbatch):
    grid = (nbatch // B_BLK,)
    return pl.pallas_call(
        _mlp_body,
        grid=grid,
        in_specs=[
            pl.BlockSpec((B_BLK, 2 * EMBED_DIM), lambda i: (i, 0)),
            pl.BlockSpec((B_BLK, NUM_GENRES), lambda i: (i, 0)),
            pl.BlockSpec((2 * EMBED_DIM + NUM_GENRES, HIDDEN_DIM),
                         lambda i: (0, 0)),
            pl.BlockSpec((1, HIDDEN_DIM), lambda i: (0, 0)),
            pl.BlockSpec((HIDDEN_DIM, 1), lambda i: (0, 0)),
            pl.BlockSpec((B_BLK,), lambda i: (i,)),
            pl.BlockSpec((1,), lambda i: (0,)),
        ],
        out_specs=pl.BlockSpec((B_BLK,), lambda i: (i,)),
        out_shape=jax.ShapeDtypeStruct((nbatch,), jnp.float32),
    )(um_rows, genre, w1, b1r, w2, bias_sum, cvec)


def _pad_bias(tab):
    flat = tab[:, 0]
    padded = jnp.pad(flat, (0, 782 * 128 - flat.shape[0]))
    return padded.reshape(782, 128)


NSPLIT = 4
HBATCH = BATCH // NSPLIT


def kernel(user_indices, movie_indices, genre_features, user_emb, movie_emb,
           user_bias_tab, movie_bias_tab, global_bias, W1, b1, W2, b2):
    uidx = user_indices.astype(jnp.int32)
    midx = movie_indices.astype(jnp.int32)
    ubt2 = _pad_bias(user_bias_tab)
    mbt2 = _pad_bias(movie_bias_tab)
    w1_bf = W1.astype(jnp.bfloat16)
    b1r = b1.reshape(1, HIDDEN_DIM)
    w2_bf = W2.astype(jnp.bfloat16)
    genre_bf = genre_features.astype(jnp.bfloat16)

    gathered = []
    for k in range(NSPLIT):
        lo = k * HBATCH
        u2 = lax.dynamic_slice_in_dim(uidx, lo, HBATCH).reshape(
            HBATCH // CHUNK, CHUNK)
        m2 = lax.dynamic_slice_in_dim(midx, lo, HBATCH).reshape(
            HBATCH // CHUNK, CHUNK)
        gathered.append(
            _sc_gather(user_emb, movie_emb, ubt2, mbt2, u2, m2, HBATCH))
    cvec = (b2 + global_bias).astype(jnp.float32)
    outs = []
    for k in range(NSPLIT):
        um_rows, bias_sum = gathered[k]
        g = lax.dynamic_slice_in_dim(genre_bf, k * HBATCH, HBATCH)
        outs.append(
            _mlp(um_rows, g, w1_bf, b1r, w2_bf, bias_sum, cvec, HBATCH))
    return jnp.concatenate(outs)


# speedup vs baseline: 1.0257x; 1.0257x over previous
import dataclasses
import functools

import jax
import jax.numpy as jnp
from jax import lax
from jax.experimental import pallas as pl
from jax.experimental.pallas import tpu as pltpu
from jax.experimental.pallas import tpu_sc as plsc

BATCH = 16384
EMBED_DIM = 128
NUM_GENRES = 32
HIDDEN_DIM = 1024

NC = 2
NS = 16
NW = NC * NS
BPW = BATCH // NW
CHUNK = 128
NCHUNK = BPW // CHUNK
NLANE = 16


def _sc_gather(user_emb, movie_emb, ubt2, mbt2, uidx2, midx2, nbatch):
    bpw = nbatch // NW
    nchunk = bpw // CHUNK
    mesh = plsc.VectorSubcoreMesh(core_axis_name="c", subcore_axis_name="s")
    out_types = (
        jax.ShapeDtypeStruct((nbatch, 2 * EMBED_DIM), jnp.float32),
        jax.ShapeDtypeStruct((nbatch,), jnp.float32),
    )

    cp = pltpu.CompilerParams()
    if "needs_layout_passes" in pltpu.CompilerParams.__dataclass_fields__:
        cp = dataclasses.replace(cp, needs_layout_passes=False)

    @functools.partial(
        pl.kernel,
        mesh=mesh,
        out_type=out_types,
        compiler_params=cp,
        scratch_types=[
            pltpu.VMEM((nchunk, CHUNK), jnp.int32),
            pltpu.VMEM((nchunk, CHUNK), jnp.int32),
            pltpu.VMEM((bpw, EMBED_DIM), jnp.float32),
            pltpu.VMEM((2, CHUNK, EMBED_DIM), jnp.float32),
            pltpu.VMEM((bpw,), jnp.float32),
            pltpu.SemaphoreType.DMA,
            pltpu.SemaphoreType.DMA,
        ],
    )
    def k(uemb, memb, ubt, mbt, uidx, midx, out_um, out_bias,
          idx_v, ridx_v, rows_v, bbuf, bias_v, sem, bsem):
        wid = lax.axis_index("s") * NC + lax.axis_index("c")
        base = wid * bpw
        row0 = wid * nchunk
        lanes = lax.iota(jnp.int32, NLANE)

        def gather_one(idx_hbm, table, bias_tab, col0, accumulate):
            pltpu.sync_copy(idx_hbm.at[pl.ds(row0, nchunk)], idx_v)
            row_cps = [
                pltpu.async_copy(
                    table.at[idx_v.at[j]],
                    rows_v.at[pl.ds(j * CHUNK, CHUNK)], sem)
                for j in range(nchunk)
            ]
            for j in range(nchunk):
                for b in range(CHUNK // NLANE):
                    iv = idx_v[j, pl.ds(b * NLANE, NLANE)]
                    ridx_v[j, pl.ds(b * NLANE, NLANE)] = iv >> 7
            nbuf = min(2, nchunk)
            bias_cps = [None] * nbuf
            for j in range(nbuf):
                bias_cps[j] = pltpu.async_copy(
                    bias_tab.at[ridx_v.at[j]], bbuf.at[j], bsem)
            for j in range(nchunk):
                bias_cps[j % nbuf].wait()
                for b in range(CHUNK // NLANE):
                    iv = idx_v[j, pl.ds(b * NLANE, NLANE)]
                    k_vec = lanes + (b * NLANE)
                    lane_vec = iv & 127
                    vals = plsc.load_gather(bbuf.at[j % nbuf], [k_vec, lane_vec])
                    dst = pl.ds(j * CHUNK + b * NLANE, NLANE)
                    if accumulate:
                        bias_v[dst] = bias_v[dst] + vals
                    else:
                        bias_v[dst] = vals
                if j + nbuf < nchunk:
                    bias_cps[j % nbuf] = pltpu.async_copy(
                        bias_tab.at[ridx_v.at[j + nbuf]], bbuf.at[j % nbuf],
                        bsem)
            for c in row_cps:
                c.wait()
            pltpu.sync_copy(
                rows_v,
                out_um.at[pl.ds(base, bpw), pl.ds(col0, EMBED_DIM)])

        gather_one(uidx, uemb, ubt, 0, False)
        gather_one(midx, memb, mbt, EMBED_DIM, True)
        pltpu.sync_copy(bias_v, out_bias.at[pl.ds(base, bpw)])

    return k(user_emb, movie_emb, ubt2, mbt2, uidx2, midx2)


B_BLK = 2048


def _mlp_body(um_ref, g_ref, w1_ref, b1_ref, w2_ref, bias_ref, c_ref, o_ref):
    um = um_ref[...].astype(jnp.bfloat16)
    f = jnp.concatenate([um, g_ref[...]], axis=1)
    h = jnp.dot(f, w1_ref[...], preferred_element_type=jnp.float32)
    h = jnp.maximum(h + b1_ref[...], 0.0).astype(jnp.bfloat16)
    s = jnp.dot(h, w2_ref[...], preferred_element_type=jnp.float32)
    o_ref[...] = s.reshape(B_BLK) + bias_ref[...] + c_ref[0]


def _mlp(um_rows, genre, w1, b1r, w2, bias_sum, cvec, nbatch):
    grid = (nbatch // B_BLK,)
    return pl.pallas_call(
        _mlp_body,
        grid=grid,
        in_specs=[
            pl.BlockSpec((B_BLK, 2 * EMBED_DIM), lambda i: (i, 0)),
            pl.BlockSpec((B_BLK, NUM_GENRES), lambda i: (i, 0)),
            pl.BlockSpec((2 * EMBED_DIM + NUM_GENRES, HIDDEN_DIM),
                         lambda i: (0, 0)),
            pl.BlockSpec((1, HIDDEN_DIM), lambda i: (0, 0)),
            pl.BlockSpec((HIDDEN_DIM, 1), lambda i: (0, 0)),
            pl.BlockSpec((B_BLK,), lambda i: (i,)),
            pl.BlockSpec((1,), lambda i: (0,)),
        ],
        out_specs=pl.BlockSpec((B_BLK,), lambda i: (i,)),
        out_shape=jax.ShapeDtypeStruct((nbatch,), jnp.float32),
    )(um_rows, genre, w1, b1r, w2, bias_sum, cvec)


def _pad_bias(tab):
    flat = tab[:, 0]
    padded = jnp.pad(flat, (0, 782 * 128 - flat.shape[0]))
    return padded.reshape(782, 128)


NSPLIT = 2
HBATCH = BATCH // NSPLIT


def kernel(user_indices, movie_indices, genre_features, user_emb, movie_emb,
           user_bias_tab, movie_bias_tab, global_bias, W1, b1, W2, b2):
    uidx = user_indices.astype(jnp.int32)
    midx = movie_indices.astype(jnp.int32)
    ubt2 = _pad_bias(user_bias_tab)
    mbt2 = _pad_bias(movie_bias_tab)
    w1_bf = W1.astype(jnp.bfloat16)
    b1r = b1.reshape(1, HIDDEN_DIM)
    w2_bf = W2.astype(jnp.bfloat16)
    genre_bf = genre_features.astype(jnp.bfloat16)

    gathered = []
    for k in range(NSPLIT):
        lo = k * HBATCH
        u2 = lax.dynamic_slice_in_dim(uidx, lo, HBATCH).reshape(
            HBATCH // CHUNK, CHUNK)
        m2 = lax.dynamic_slice_in_dim(midx, lo, HBATCH).reshape(
            HBATCH // CHUNK, CHUNK)
        gathered.append(
            _sc_gather(user_emb, movie_emb, ubt2, mbt2, u2, m2, HBATCH))
    cvec = (b2 + global_bias).astype(jnp.float32)
    outs = []
    for k in range(NSPLIT):
        um_rows, bias_sum = gathered[k]
        g = lax.dynamic_slice_in_dim(genre_bf, k * HBATCH, HBATCH)
        outs.append(
            _mlp(um_rows, g, w1_bf, b1r, w2_bf, bias_sum, cvec, HBATCH))
    return jnp.concatenate(outs)


# speedup vs baseline: 1.0818x; 1.0547x over previous
import dataclasses
import functools

import jax
import jax.numpy as jnp
from jax import lax
from jax.experimental import pallas as pl
from jax.experimental.pallas import tpu as pltpu
from jax.experimental.pallas import tpu_sc as plsc

BATCH = 16384
EMBED_DIM = 128
NUM_GENRES = 32
HIDDEN_DIM = 1024

NC = 2
NS = 16
NW = NC * NS
BPW = BATCH // NW
CHUNK = 128
NCHUNK = BPW // CHUNK
NLANE = 16


BIAS_PAD = 100096


def _sc_gather(user_emb, movie_emb, ubt, mbt, uidx2, midx2, nbatch):
    bpw = nbatch // NW
    nchunk = bpw // CHUNK
    seg = BIAS_PAD // NS
    mesh = plsc.VectorSubcoreMesh(core_axis_name="c", subcore_axis_name="s")
    out_types = (
        jax.ShapeDtypeStruct((nbatch, 2 * EMBED_DIM), jnp.float32),
        jax.ShapeDtypeStruct((nbatch,), jnp.float32),
    )

    cp = pltpu.CompilerParams()
    if "needs_layout_passes" in pltpu.CompilerParams.__dataclass_fields__:
        cp = dataclasses.replace(cp, needs_layout_passes=False)

    @functools.partial(
        pl.kernel,
        mesh=mesh,
        out_type=out_types,
        compiler_params=cp,
        scratch_types=[
            pltpu.VMEM((nchunk, CHUNK), jnp.int32),
            pltpu.VMEM((nchunk, CHUNK), jnp.int32),
            pltpu.VMEM((bpw, EMBED_DIM), jnp.float32),
            pltpu.VMEM((bpw, EMBED_DIM), jnp.float32),
            pltpu.VMEM((nchunk, CHUNK), jnp.float32),
            pltpu.VMEM((nchunk, CHUNK), jnp.float32),
            pltpu.VMEM((bpw,), jnp.float32),
            pltpu.VMEM((seg,), jnp.float32),
            pltpu.VMEM((seg,), jnp.float32),
            pltpu.VMEM_SHARED((BIAS_PAD,), jnp.float32),
            pltpu.VMEM_SHARED((BIAS_PAD,), jnp.float32),
            pltpu.SemaphoreType.DMA,
            pltpu.SemaphoreType.DMA,
            pltpu.SemaphoreType.DMA,
        ],
    )
    def k(uemb, memb, ubt_r, mbt_r, uidx, midx, out_um, out_bias,
          uidx_v, midx_v, urows_v, mrows_v, ubuf, mbuf, bias_v,
          ustage_v, mstage_v, ub_sh, mb_sh, sem, ssem, bsem):
        sid = lax.axis_index("s")
        wid = sid * NC + lax.axis_index("c")
        base = wid * bpw
        row0 = wid * nchunk

        pltpu.sync_copy(uidx.at[pl.ds(row0, nchunk)], uidx_v)
        pltpu.sync_copy(midx.at[pl.ds(row0, nchunk)], midx_v)
        row_cps = [
            pltpu.async_copy(uemb.at[uidx_v.at[j]],
                             urows_v.at[pl.ds(j * CHUNK, CHUNK)], sem)
            for j in range(nchunk)
        ] + [
            pltpu.async_copy(memb.at[midx_v.at[j]],
                             mrows_v.at[pl.ds(j * CHUNK, CHUNK)], sem)
            for j in range(nchunk)
        ]
        off = sid * seg
        pltpu.sync_copy(ubt_r.at[pl.ds(off, seg)], ustage_v)
        pltpu.sync_copy(mbt_r.at[pl.ds(off, seg)], mstage_v)
        st = [
            pltpu.async_copy(ustage_v, ub_sh.at[pl.ds(off, seg)], ssem),
            pltpu.async_copy(mstage_v, mb_sh.at[pl.ds(off, seg)], ssem),
        ]
        for c in st:
            c.wait()
        plsc.subcore_barrier()
        bias_cps = [
            pltpu.async_copy(ub_sh.at[uidx_v.at[j]], ubuf.at[j], bsem)
            for j in range(nchunk)
        ] + [
            pltpu.async_copy(mb_sh.at[midx_v.at[j]], mbuf.at[j], bsem)
            for j in range(nchunk)
        ]
        for c in row_cps:
            c.wait()
        pltpu.sync_copy(urows_v,
                        out_um.at[pl.ds(base, bpw), pl.ds(0, EMBED_DIM)])
        pltpu.sync_copy(
            mrows_v,
            out_um.at[pl.ds(base, bpw), pl.ds(EMBED_DIM, EMBED_DIM)])
        for c in bias_cps:
            c.wait()
        for j in range(nchunk):
            for b in range(CHUNK // NLANE):
                src = pl.ds(b * NLANE, NLANE)
                dst = pl.ds(j * CHUNK + b * NLANE, NLANE)
                bias_v[dst] = ubuf[j, src] + mbuf[j, src]
        pltpu.sync_copy(bias_v, out_bias.at[pl.ds(base, bpw)])

    return k(user_emb, movie_emb, ubt, mbt, uidx2, midx2)


B_BLK = 2048


def _mlp_body(um_ref, g_ref, w1_ref, b1_ref, w2_ref, bias_ref, c_ref, o_ref):
    um = um_ref[...].astype(jnp.bfloat16)
    f = jnp.concatenate([um, g_ref[...]], axis=1)
    h = jnp.dot(f, w1_ref[...], preferred_element_type=jnp.float32)
    h = jnp.maximum(h + b1_ref[...], 0.0).astype(jnp.bfloat16)
    s = jnp.dot(h, w2_ref[...], preferred_element_type=jnp.float32)
    o_ref[...] = s.reshape(B_BLK) + bias_ref[...] + c_ref[0]


def _mlp(um_rows, genre, w1, b1r, w2, bias_sum, cvec, nbatch):
    grid = (nbatch // B_BLK,)
    return pl.pallas_call(
        _mlp_body,
        grid=grid,
        in_specs=[
            pl.BlockSpec((B_BLK, 2 * EMBED_DIM), lambda i: (i, 0)),
            pl.BlockSpec((B_BLK, NUM_GENRES), lambda i: (i, 0)),
            pl.BlockSpec((2 * EMBED_DIM + NUM_GENRES, HIDDEN_DIM),
                         lambda i: (0, 0)),
            pl.BlockSpec((1, HIDDEN_DIM), lambda i: (0, 0)),
            pl.BlockSpec((HIDDEN_DIM, 1), lambda i: (0, 0)),
            pl.BlockSpec((B_BLK,), lambda i: (i,)),
            pl.BlockSpec((1,), lambda i: (0,)),
        ],
        out_specs=pl.BlockSpec((B_BLK,), lambda i: (i,)),
        out_shape=jax.ShapeDtypeStruct((nbatch,), jnp.float32),
    )(um_rows, genre, w1, b1r, w2, bias_sum, cvec)


def _pad_bias(tab):
    return jnp.pad(tab[:, 0], (0, BIAS_PAD - tab.shape[0]))


NSPLIT = 2
HBATCH = BATCH // NSPLIT


def kernel(user_indices, movie_indices, genre_features, user_emb, movie_emb,
           user_bias_tab, movie_bias_tab, global_bias, W1, b1, W2, b2):
    uidx = user_indices.astype(jnp.int32)
    midx = movie_indices.astype(jnp.int32)
    ubt2 = _pad_bias(user_bias_tab)
    mbt2 = _pad_bias(movie_bias_tab)
    w1_bf = W1.astype(jnp.bfloat16)
    b1r = b1.reshape(1, HIDDEN_DIM)
    w2_bf = W2.astype(jnp.bfloat16)
    genre_bf = genre_features.astype(jnp.bfloat16)

    gathered = []
    for k in range(NSPLIT):
        lo = k * HBATCH
        u2 = lax.dynamic_slice_in_dim(uidx, lo, HBATCH).reshape(
            HBATCH // CHUNK, CHUNK)
        m2 = lax.dynamic_slice_in_dim(midx, lo, HBATCH).reshape(
            HBATCH // CHUNK, CHUNK)
        gathered.append(
            _sc_gather(user_emb, movie_emb, ubt2, mbt2, u2, m2, HBATCH))
    cvec = (b2 + global_bias).astype(jnp.float32)
    outs = []
    for k in range(NSPLIT):
        um_rows, bias_sum = gathered[k]
        g = lax.dynamic_slice_in_dim(genre_bf, k * HBATCH, HBATCH)
        outs.append(
            _mlp(um_rows, g, w1_bf, b1r, w2_bf, bias_sum, cvec, HBATCH))
    return jnp.concatenate(outs)


# speedup vs baseline: 1.0840x; 1.0020x over previous
import dataclasses
import functools

import jax
import jax.numpy as jnp
from jax import lax
from jax.experimental import pallas as pl
from jax.experimental.pallas import tpu as pltpu
from jax.experimental.pallas import tpu_sc as plsc

BATCH = 16384
EMBED_DIM = 128
NUM_GENRES = 32
HIDDEN_DIM = 1024

NC = 2
NS = 16
NW = NC * NS
BPW = BATCH // NW
CHUNK = 128
NCHUNK = BPW // CHUNK
NLANE = 16


BIAS_PAD = 100096


def _sc_gather(user_emb, movie_emb, ubt, mbt, uidx2, midx2, nbatch):
    bpw = nbatch // NW
    nchunk = bpw // CHUNK
    seg = BIAS_PAD // NS
    mesh = plsc.VectorSubcoreMesh(core_axis_name="c", subcore_axis_name="s")
    out_types = (
        jax.ShapeDtypeStruct((nbatch, 2 * EMBED_DIM), jnp.float32),
        jax.ShapeDtypeStruct((nbatch,), jnp.float32),
    )

    cp = pltpu.CompilerParams()
    if "needs_layout_passes" in pltpu.CompilerParams.__dataclass_fields__:
        cp = dataclasses.replace(cp, needs_layout_passes=False)

    @functools.partial(
        pl.kernel,
        mesh=mesh,
        out_type=out_types,
        compiler_params=cp,
        scratch_types=[
            pltpu.VMEM((nchunk, CHUNK), jnp.int32),
            pltpu.VMEM((nchunk, CHUNK), jnp.int32),
            pltpu.VMEM((bpw, EMBED_DIM), jnp.float32),
            pltpu.VMEM((bpw, EMBED_DIM), jnp.float32),
            pltpu.VMEM((nchunk, CHUNK), jnp.float32),
            pltpu.VMEM((nchunk, CHUNK), jnp.float32),
            pltpu.VMEM((bpw,), jnp.float32),
            pltpu.VMEM((seg,), jnp.float32),
            pltpu.VMEM((seg,), jnp.float32),
            pltpu.VMEM_SHARED((BIAS_PAD,), jnp.float32),
            pltpu.VMEM_SHARED((BIAS_PAD,), jnp.float32),
            pltpu.SemaphoreType.DMA,
            pltpu.SemaphoreType.DMA,
            pltpu.SemaphoreType.DMA,
        ],
    )
    def k(uemb, memb, ubt_r, mbt_r, uidx, midx, out_um, out_bias,
          uidx_v, midx_v, urows_v, mrows_v, ubuf, mbuf, bias_v,
          ustage_v, mstage_v, ub_sh, mb_sh, sem, ssem, bsem):
        sid = lax.axis_index("s")
        wid = sid * NC + lax.axis_index("c")
        base = wid * bpw
        row0 = wid * nchunk

        pltpu.sync_copy(uidx.at[pl.ds(row0, nchunk)], uidx_v)
        pltpu.sync_copy(midx.at[pl.ds(row0, nchunk)], midx_v)
        row_cps = [
            pltpu.async_copy(uemb.at[uidx_v.at[j]],
                             urows_v.at[pl.ds(j * CHUNK, CHUNK)], sem)
            for j in range(nchunk)
        ] + [
            pltpu.async_copy(memb.at[midx_v.at[j]],
                             mrows_v.at[pl.ds(j * CHUNK, CHUNK)], sem)
            for j in range(nchunk)
        ]
        off = sid * seg
        pltpu.sync_copy(ubt_r.at[pl.ds(off, seg)], ustage_v)
        pltpu.sync_copy(mbt_r.at[pl.ds(off, seg)], mstage_v)
        st = [
            pltpu.async_copy(ustage_v, ub_sh.at[pl.ds(off, seg)], ssem),
            pltpu.async_copy(mstage_v, mb_sh.at[pl.ds(off, seg)], ssem),
        ]
        for c in st:
            c.wait()
        plsc.subcore_barrier()
        bias_cps = [
            pltpu.async_copy(ub_sh.at[uidx_v.at[j]], ubuf.at[j], bsem)
            for j in range(nchunk)
        ] + [
            pltpu.async_copy(mb_sh.at[midx_v.at[j]], mbuf.at[j], bsem)
            for j in range(nchunk)
        ]
        for c in row_cps:
            c.wait()
        pltpu.sync_copy(urows_v,
                        out_um.at[pl.ds(base, bpw), pl.ds(0, EMBED_DIM)])
        pltpu.sync_copy(
            mrows_v,
            out_um.at[pl.ds(base, bpw), pl.ds(EMBED_DIM, EMBED_DIM)])
        for c in bias_cps:
            c.wait()
        for j in range(nchunk):
            for b in range(CHUNK // NLANE):
                src = pl.ds(b * NLANE, NLANE)
                dst = pl.ds(j * CHUNK + b * NLANE, NLANE)
                bias_v[dst] = ubuf[j, src] + mbuf[j, src]
        pltpu.sync_copy(bias_v, out_bias.at[pl.ds(base, bpw)])

    return k(user_emb, movie_emb, ubt, mbt, uidx2, midx2)


B_BLK = 4096


def _mlp_body(um_ref, g_ref, w1_ref, b1_ref, w2_ref, bias_ref, c_ref, o_ref):
    um = um_ref[...].astype(jnp.bfloat16)
    f = jnp.concatenate([um, g_ref[...]], axis=1)
    h = jnp.dot(f, w1_ref[...], preferred_element_type=jnp.float32)
    h = jnp.maximum(h + b1_ref[...], 0.0).astype(jnp.bfloat16)
    s = jnp.dot(h, w2_ref[...], preferred_element_type=jnp.float32)
    o_ref[...] = s.reshape(B_BLK) + bias_ref[...] + c_ref[0]


def _mlp(um_rows, genre, w1, b1r, w2, bias_sum, cvec, nbatch):
    grid = (nbatch // B_BLK,)
    return pl.pallas_call(
        _mlp_body,
        grid=grid,
        in_specs=[
            pl.BlockSpec((B_BLK, 2 * EMBED_DIM), lambda i: (i, 0)),
            pl.BlockSpec((B_BLK, NUM_GENRES), lambda i: (i, 0)),
            pl.BlockSpec((2 * EMBED_DIM + NUM_GENRES, HIDDEN_DIM),
                         lambda i: (0, 0)),
            pl.BlockSpec((1, HIDDEN_DIM), lambda i: (0, 0)),
            pl.BlockSpec((HIDDEN_DIM, 1), lambda i: (0, 0)),
            pl.BlockSpec((B_BLK,), lambda i: (i,)),
            pl.BlockSpec((1,), lambda i: (0,)),
        ],
        out_specs=pl.BlockSpec((B_BLK,), lambda i: (i,)),
        out_shape=jax.ShapeDtypeStruct((nbatch,), jnp.float32),
    )(um_rows, genre, w1, b1r, w2, bias_sum, cvec)


def _pad_bias(tab):
    return jnp.pad(tab[:, 0], (0, BIAS_PAD - tab.shape[0]))


NSPLIT = 2
HBATCH = BATCH // NSPLIT


def kernel(user_indices, movie_indices, genre_features, user_emb, movie_emb,
           user_bias_tab, movie_bias_tab, global_bias, W1, b1, W2, b2):
    uidx = user_indices.astype(jnp.int32)
    midx = movie_indices.astype(jnp.int32)
    ubt2 = _pad_bias(user_bias_tab)
    mbt2 = _pad_bias(movie_bias_tab)
    w1_bf = W1.astype(jnp.bfloat16)
    b1r = b1.reshape(1, HIDDEN_DIM)
    w2_bf = W2.astype(jnp.bfloat16)
    genre_bf = genre_features.astype(jnp.bfloat16)

    gathered = []
    for k in range(NSPLIT):
        lo = k * HBATCH
        u2 = lax.dynamic_slice_in_dim(uidx, lo, HBATCH).reshape(
            HBATCH // CHUNK, CHUNK)
        m2 = lax.dynamic_slice_in_dim(midx, lo, HBATCH).reshape(
            HBATCH // CHUNK, CHUNK)
        gathered.append(
            _sc_gather(user_emb, movie_emb, ubt2, mbt2, u2, m2, HBATCH))
    cvec = (b2 + global_bias).astype(jnp.float32)
    outs = []
    for k in range(NSPLIT):
        um_rows, bias_sum = gathered[k]
        g = lax.dynamic_slice_in_dim(genre_bf, k * HBATCH, HBATCH)
        outs.append(
            _mlp(um_rows, g, w1_bf, b1r, w2_bf, bias_sum, cvec, HBATCH))
    return jnp.concatenate(outs)
